# baseline (speedup 1.0000x reference)
import jax
import jax.numpy as jnp
from jax import lax
from jax.experimental import pallas as pl
from jax.experimental.pallas import tpu as pltpu
from jax.experimental.pallas import tpu_sc as plsc

N = 10000
E = 320000
D = 128
G = 64

NC, NS = 2, 16
NW = NC * NS
NP = 10240
RPT = NP // NS
CH, C = 80, 128
EW = CH * C
EP = NW * EW
EB = EP // 128
CR = 64
FPT = D // NW

_f32 = jnp.float32
_i32 = jnp.int32


def _mesh():
    return plsc.VectorSubcoreMesh(core_axis_name="c", subcore_axis_name="s",
                                  num_cores=NC, num_subcores=NS)


def _pre_body(dsts, ews, srcs, dinv_out, coef_out,
              dstv, ewv, srcv, degp, sumb, dinvc, dinv_full,
              coefv, sh_deg, sh_dinv):
    c = lax.axis_index("c")
    s = lax.axis_index("s")

    def _zd(i, carry):
        degp[pl.ds(i * 16, 16)] = jnp.zeros((16,), _f32)
        return carry
    lax.fori_loop(0, NP // 16, _zd, None)

    def _slice(k):
        pltpu.sync_copy(dsts.at[k], dstv)
        pltpu.sync_copy(ews.at[k], ewv)
        def _g(g, carry):
            r, col = g // 8, (g % 8) * 16
            d16 = dstv[r, pl.ds(col, 16)]
            w16 = ewv[r, pl.ds(col, 16)]
            plsc.addupdate_scatter(degp, [d16], w16)
            return carry
        lax.fori_loop(0, CH * 8, _g, None)
    _slice(2 * s)
    _slice(2 * s + 1)

    pltpu.sync_copy(degp, sh_deg.at[s])
    plsc.subcore_barrier()

    pltpu.sync_copy(sh_deg.at[:, pl.ds(s * RPT, RPT)], sumb)
    def _rs(v, carry):
        dsum = jnp.zeros((16,), _f32)
        for r in range(NS):
            dsum = dsum + sumb[r, pl.ds(v * 16, 16)]
        dsum = dsum + 1.0
        i = plsc.bitcast(dsum, _i32)
        i = 0x5F3759DF - lax.shift_right_logical(i, 1)
        y = plsc.bitcast(i, _f32)
        for _ in range(3):
            y = y * (1.5 - 0.5 * dsum * y * y)
        dinvc[pl.ds(v * 16, 16)] = y
        return carry
    lax.fori_loop(0, 40, _rs, None)
    pltpu.sync_copy(dinvc, sh_dinv.at[pl.ds(s * RPT, RPT)])

    @pl.when(c == 0)
    def _():
        pltpu.sync_copy(dinvc, dinv_out.at[pl.ds(s * RPT, RPT)])
    plsc.subcore_barrier()

    pltpu.sync_copy(sh_dinv, dinv_full)
    wid = c * NS + s
    pltpu.sync_copy(srcs.at[wid], srcv)
    pltpu.sync_copy(dsts.at[wid], dstv)
    pltpu.sync_copy(ews.at[wid], ewv)
    def _ce(g, carry):
        r, col = g // 8, (g % 8) * 16
        s16 = srcv[r, pl.ds(col, 16)]
        d16 = dstv[r, pl.ds(col, 16)]
        w16 = ewv[r, pl.ds(col, 16)]
        cc = w16 * plsc.load_gather(dinv_full, [s16]) * plsc.load_gather(dinv_full, [d16])
        coefv[r, pl.ds(col, 16)] = cc
        return carry
    lax.fori_loop(0, CH * 8, _ce, None)
    pltpu.sync_copy(coefv, coef_out.at[wid])


def _pre(dsts, ews, srcs):
    f = pl.kernel(
        _pre_body,
        out_type=(jax.ShapeDtypeStruct((NP,), _f32),
                  jax.ShapeDtypeStruct((NW, CH, C), _f32)),
        mesh=_mesh(),
        compiler_params=pltpu.CompilerParams(needs_layout_passes=False),
        scratch_types=[
            pltpu.VMEM((CH, C), _i32),
            pltpu.VMEM((CH, C), _f32),
            pltpu.VMEM((CH, C), _i32),
            pltpu.VMEM((NP,), _f32),
            pltpu.VMEM((NS, RPT), _f32),
            pltpu.VMEM((RPT,), _f32),
            pltpu.VMEM((NP,), _f32),
            pltpu.VMEM((CH, C), _f32),
            pltpu.VMEM_SHARED((NS, NP), _f32),
            pltpu.VMEM_SHARED((NP,), _f32),
        ],
    )
    return f(dsts, ews, srcs)


def _gprop_body(hT, srcs, dsts, coefs, accT_out,
                h0, h1, h2, h3, a0, a1, a2, a3,
                sb0, db0, cb0, sb1, db1, cb1, sg0, sg1):
    c = lax.axis_index("c")
    s = lax.axis_index("s")
    wid = c * NS + s
    fb = FPT * wid

    hs = (h0, h1, h2, h3)
    ac = (a0, a1, a2, a3)
    for f in range(FPT):
        pltpu.sync_copy(hT.at[fb + f], hs[f])
    def _za(i, carry):
        z = jnp.zeros((16,), _f32)
        for f in range(FPT):
            ac[f][pl.ds(i * 16, 16)] = z
        return carry
    lax.fori_loop(0, NP // 16, _za, None)

    def _issue(j, sb, db, cb, sem):
        off = pl.multiple_of(j * CR, 8)
        pltpu.async_copy(srcs.at[pl.ds(off, CR), :], sb, sem)
        pltpu.async_copy(dsts.at[pl.ds(off, CR), :], db, sem)
        pltpu.async_copy(coefs.at[pl.ds(off, CR), :], cb, sem)

    def _drain(sb, db, cb, sem):
        pltpu.make_async_copy(srcs.at[pl.ds(0, CR), :], sb, sem).wait()
        pltpu.make_async_copy(dsts.at[pl.ds(0, CR), :], db, sem).wait()
        pltpu.make_async_copy(coefs.at[pl.ds(0, CR), :], cb, sem).wait()

    def _process(sb, db, cb):
        def _row(r, carry):
            for g8 in range(8):
                col = g8 * 16
                s16 = sb[r, pl.ds(col, 16)]
                d16 = db[r, pl.ds(col, 16)]
                c16 = cb[r, pl.ds(col, 16)]
                for f in range(FPT):
                    v = plsc.load_gather(hs[f], [s16]) * c16
                    plsc.addupdate_scatter(ac[f], [d16], v)
            return carry
        lax.fori_loop(0, CR, _row, None)

    nch = EB // CR
    _issue(0, sb0, db0, cb0, sg0)

    def _pair(t, carry):
        j0 = 2 * t
        _issue(j0 + 1, sb1, db1, cb1, sg1)
        _drain(sb0, db0, cb0, sg0)
        _process(sb0, db0, cb0)
        j2 = jnp.minimum(j0 + 2, nch - 2)
        _issue(j2, sb0, db0, cb0, sg0)
        _drain(sb1, db1, cb1, sg1)
        _process(sb1, db1, cb1)
        return carry
    lax.fori_loop(0, nch // 2, _pair, None)
    _drain(sb0, db0, cb0, sg0)

    for f in range(FPT):
        pltpu.sync_copy(ac[f], accT_out.at[fb + f])


def _gprop(hT, srcs_r, dsts_r, coefs_r):
    f = pl.kernel(
        _gprop_body,
        out_type=jax.ShapeDtypeStruct((D, NP), _f32),
        mesh=_mesh(),
        compiler_params=pltpu.CompilerParams(needs_layout_passes=False),
        scratch_types=(
            [pltpu.VMEM((NP,), _f32)] * 4
            + [pltpu.VMEM((NP,), _f32)] * 4
            + [pltpu.VMEM((CR, 128), _i32),
               pltpu.VMEM((CR, 128), _i32),
               pltpu.VMEM((CR, 128), _f32),
               pltpu.VMEM((CR, 128), _i32),
               pltpu.VMEM((CR, 128), _i32),
               pltpu.VMEM((CR, 128), _f32),
               pltpu.SemaphoreType.DMA,
               pltpu.SemaphoreType.DMA]
        ),
    )
    return f(hT, srcs_r, dsts_r, coefs_r)


_BM = 512


def _mm0_body(x_ref, w_ref, o_ref):
    o_ref[...] = lax.dot_general(w_ref[...], x_ref[...],
                                 (((0,), (1,)), ((), ())),
                                 preferred_element_type=_f32)


def _mm0(xp, W):
    return pl.pallas_call(
        _mm0_body,
        grid=(NP // _BM,),
        in_specs=[pl.BlockSpec((_BM, D), lambda i: (i, 0)),
                  pl.BlockSpec((D, D), lambda i: (0, 0))],
        out_specs=pl.BlockSpec((D, _BM), lambda i: (0, i)),
        out_shape=jax.ShapeDtypeStruct((D, NP), _f32),
    )(xp, W)


def _mid_body(acc_ref, h_ref, di_ref, b_ref, w_ref, o_ref):
    d = di_ref[...]
    xb = acc_ref[...] + d * d * h_ref[...] + b_ref[...]
    xb = jnp.maximum(xb, 0.0)
    o_ref[...] = lax.dot_general(w_ref[...], xb, (((0,), (0,)), ((), ())),
                                 preferred_element_type=_f32)


def _mid(accT, hT, dinvT, bc, W):
    return pl.pallas_call(
        _mid_body,
        grid=(NP // _BM,),
        in_specs=[pl.BlockSpec((D, _BM), lambda i: (0, i)),
                  pl.BlockSpec((D, _BM), lambda i: (0, i)),
                  pl.BlockSpec((1, _BM), lambda i: (0, i)),
                  pl.BlockSpec((D, 1), lambda i: (0, 0)),
                  pl.BlockSpec((D, D), lambda i: (0, 0))],
        out_specs=pl.BlockSpec((D, _BM), lambda i: (0, i)),
        out_shape=jax.ShapeDtypeStruct((D, NP), _f32),
    )(accT, hT, dinvT, bc, W)


_BM7 = 512


def _read_body(acc_ref, h_ref, di_ref, b_ref, r0_ref, rb0_ref, r1_ref,
               rb1_ref, bt_ref, o_ref, acc_s, acc_c):
    i = pl.program_id(0)

    @pl.when(i == 0)
    def _():
        acc_s[...] = jnp.zeros_like(acc_s)
        acc_c[...] = jnp.zeros_like(acc_c)

    d = di_ref[...]
    x2 = acc_ref[...] + d * d * h_ref[...] + b_ref[...]
    t = jnp.maximum(
        lax.dot_general(r0_ref[...], x2, (((0,), (0,)), ((), ())),
                        preferred_element_type=_f32) + rb0_ref[...], 0.0)
    r = lax.dot_general(r1_ref[...], t, (((0,), (0,)), ((), ())),
                        preferred_element_type=_f32) + rb1_ref[0, 0]
    oh = (bt_ref[...] == lax.broadcasted_iota(_i32, (128, _BM7), 0)).astype(_f32)
    acc_s[...] += lax.dot_general(oh, r, (((1,), (1,)), ((), ())),
                                  preferred_element_type=_f32)
    acc_c[...] += lax.dot_general(oh, jnp.ones((1, _BM7), _f32),
                                  (((1,), (1,)), ((), ())),
                                  preferred_element_type=_f32)

    @pl.when(i == pl.num_programs(0) - 1)
    def _():
        o_ref[...] = acc_s[...] / jnp.maximum(acc_c[...], 1.0)


def _read(accT, hT, dinvT, bc, R0p, rb0c, R1p, rb1p, btT):
    return pl.pallas_call(
        _read_body,
        grid=(NP // _BM7,),
        in_specs=[pl.BlockSpec((D, _BM7), lambda i: (0, i)),
                  pl.BlockSpec((D, _BM7), lambda i: (0, i)),
                  pl.BlockSpec((1, _BM7), lambda i: (0, i)),
                  pl.BlockSpec((D, 1), lambda i: (0, 0)),
                  pl.BlockSpec((D, D), lambda i: (0, 0)),
                  pl.BlockSpec((D, 1), lambda i: (0, 0)),
                  pl.BlockSpec((D, 1), lambda i: (0, 0)),
                  pl.BlockSpec((1, 1), lambda i: (0, 0)),
                  pl.BlockSpec((1, _BM7), lambda i: (0, i))],
        out_specs=pl.BlockSpec((128, 1), lambda i: (0, 0)),
        out_shape=jax.ShapeDtypeStruct((128, 1), _f32),
        scratch_shapes=[pltpu.VMEM((128, 1), _f32),
                        pltpu.VMEM((128, 1), _f32)],
    )(accT, hT, dinvT, bc, R0p, rb0c, R1p, rb1p, btT)


def kernel(x, edge_index, edge_attr, batch, W0, b0, W1, b1, W2, b2, R0, rb0, R1, rb1):
    src = edge_index[0]
    dst = edge_index[1]
    src_p = jnp.pad(src, (0, EP - E))
    dst_p = jnp.pad(dst, (0, EP - E))
    ew_p = jnp.pad(edge_attr, (0, EP - E))
    srcs = src_p.reshape(NW, CH, C)
    dsts = dst_p.reshape(NW, CH, C)
    ews = ew_p.reshape(NW, CH, C)
    srcs_r = src_p.reshape(EB, 128)
    dsts_r = dst_p.reshape(EB, 128)
    xp = jnp.pad(x, ((0, NP - N), (0, 0)))
    btT = jnp.pad(batch, (0, NP - N), constant_values=127).reshape(1, NP)
    R0p = jnp.pad(R0, ((0, 0), (0, 128 - R0.shape[1])))
    rb0c = jnp.pad(rb0, (0, 128 - rb0.shape[0])).reshape(128, 1)
    R1p = jnp.pad(R1, ((0, 128 - R1.shape[0]), (0, 0)))
    rb1p = rb1.reshape(1, 1)
    b0c = b0.reshape(D, 1)
    b1c = b1.reshape(D, 1)
    b2c = b2.reshape(D, 1)

    dinv, coef = _pre(dsts, ews, srcs)
    dinvT = dinv.reshape(1, NP)
    coefs_r = coef.reshape(EB, 128)

    h0T = _mm0(xp, W0)
    accT = _gprop(h0T, srcs_r, dsts_r, coefs_r)
    h1T = _mid(accT, h0T, dinvT, b0c, W1)
    accT = _gprop(h1T, srcs_r, dsts_r, coefs_r)
    h2T = _mid(accT, h1T, dinvT, b1c, W2)
    accT = _gprop(h2T, srcs_r, dsts_r, coefs_r)
    outp = _read(accT, h2T, dinvT, b2c, R0p, rb0c, R1p, rb1p, btT)
    return outp[:G]


# speedup vs baseline: 1.0585x; 1.0585x over previous
import jax
import jax.numpy as jnp
from jax import lax
from jax.experimental import pallas as pl
from jax.experimental.pallas import tpu as pltpu
from jax.experimental.pallas import tpu_sc as plsc

N = 10000
E = 320000
D = 128
G = 64

NC, NS = 2, 16
NW = NC * NS
NP = 10240
RPT = NP // NS
CH, C = 80, 128
QC = 8
EW = CH * C
EP = NW * EW
EB = EP // 128
N0 = 48
N1 = 160 - N0
R0B = NS * N0

_f32 = jnp.float32
_i32 = jnp.int32


def _mesh():
    return plsc.VectorSubcoreMesh(core_axis_name="c", subcore_axis_name="s",
                                  num_cores=NC, num_subcores=NS)


def _zero16(ref, ngroups):
    def _z(g, carry):
        ref[g // 8, pl.ds((g % 8) * 16, 16)] = jnp.zeros((16,), _f32)
        return carry
    lax.fori_loop(0, ngroups, _z, None)


def _pre_body(dsts, ews, srcs, dinv_out, coef_out,
              dstv, ewv, srcv, degp, sumb, dinvc, dinv_full,
              coefv, sh_deg, sh_dinv):
    c = lax.axis_index("c")
    s = lax.axis_index("s")

    def _zd(i, carry):
        degp[pl.ds(i * 16, 16)] = jnp.zeros((16,), _f32)
        return carry
    lax.fori_loop(0, NP // 16, _zd, None)

    def _slice(k):
        pltpu.sync_copy(dsts.at[k], dstv)
        pltpu.sync_copy(ews.at[k], ewv)
        def _g(g, carry):
            r, col = g // 8, (g % 8) * 16
            d16 = dstv[r, pl.ds(col, 16)]
            w16 = ewv[r, pl.ds(col, 16)]
            plsc.addupdate_scatter(degp, [d16], w16)
            return carry
        lax.fori_loop(0, CH * 8, _g, None)
    _slice(2 * s)
    _slice(2 * s + 1)

    pltpu.sync_copy(degp, sh_deg.at[s])
    plsc.subcore_barrier()

    pltpu.sync_copy(sh_deg.at[:, pl.ds(s * RPT, RPT)], sumb)
    def _rs(v, carry):
        dsum = jnp.zeros((16,), _f32)
        for r in range(NS):
            dsum = dsum + sumb[r, pl.ds(v * 16, 16)]
        dsum = dsum + 1.0
        i = plsc.bitcast(dsum, _i32)
        i = 0x5F3759DF - lax.shift_right_logical(i, 1)
        y = plsc.bitcast(i, _f32)
        for _ in range(3):
            y = y * (1.5 - 0.5 * dsum * y * y)
        dinvc[pl.ds(v * 16, 16)] = y
        return carry
    lax.fori_loop(0, 40, _rs, None)
    pltpu.sync_copy(dinvc, sh_dinv.at[pl.ds(s * RPT, RPT)])

    @pl.when(c == 0)
    def _():
        pltpu.sync_copy(dinvc, dinv_out.at[pl.ds(s * RPT, RPT)])
    plsc.subcore_barrier()

    pltpu.sync_copy(sh_dinv, dinv_full)
    wid = c * NS + s
    pltpu.sync_copy(srcs.at[wid], srcv)
    pltpu.sync_copy(dsts.at[wid], dstv)
    pltpu.sync_copy(ews.at[wid], ewv)
    def _ce(g, carry):
        r, col = g // 8, (g % 8) * 16
        s16 = srcv[r, pl.ds(col, 16)]
        d16 = dstv[r, pl.ds(col, 16)]
        w16 = ewv[r, pl.ds(col, 16)]
        cc = w16 * plsc.load_gather(dinv_full, [s16]) * plsc.load_gather(dinv_full, [d16])
        coefv[r, pl.ds(col, 16)] = cc
        return carry
    lax.fori_loop(0, CH * 8, _ce, None)
    pltpu.sync_copy(coefv, coef_out.at[wid])


def _pre(dsts, ews, srcs):
    f = pl.kernel(
        _pre_body,
        out_type=(jax.ShapeDtypeStruct((NP,), _f32),
                  jax.ShapeDtypeStruct((NW, CH, C), _f32)),
        mesh=_mesh(),
        compiler_params=pltpu.CompilerParams(needs_layout_passes=False),
        scratch_types=[
            pltpu.VMEM((CH, C), _i32),
            pltpu.VMEM((CH, C), _f32),
            pltpu.VMEM((CH, C), _i32),
            pltpu.VMEM((NP,), _f32),
            pltpu.VMEM((NS, RPT), _f32),
            pltpu.VMEM((RPT,), _f32),
            pltpu.VMEM((NP,), _f32),
            pltpu.VMEM((CH, C), _f32),
            pltpu.VMEM_SHARED((NS, NP), _f32),
            pltpu.VMEM_SHARED((NP,), _f32),
        ],
    )
    return f(dsts, ews, srcs)


def _prop_body(h, srcs, dsts, coefs, acc_out,
               srcv, dstv, coefv, rows0, rows1, acc_sh, sem0, sem1):
    c = lax.axis_index("c")
    s = lax.axis_index("s")

    _zero16(rows0, C * 8)
    for k in range(RPT // C):
        pltpu.sync_copy(rows0, acc_sh.at[pl.ds(s * RPT + k * C, C), :])
    plsc.subcore_barrier()

    def _scale(rows, j):
        def _grp(g8, c2):
            c16 = coefv[j, pl.ds(g8 * 16, 16)]
            for r16 in range(16):
                e = g8 * 16 + r16
                bc = c16.at[jnp.full((16,), r16, _i32)].get(
                    mode="promise_in_bounds")
                for k in range(8):
                    sl = pl.ds(k * 16, 16)
                    rows[e, sl] = rows[e, sl] * bc
            return c2
        lax.fori_loop(0, 8, _grp, None)

    ng = jnp.where(c == 0, N0 // QC, N1 // QC)
    base = jnp.where(c == 0, s * N0, R0B + s * N1)

    def _quarter(q, carry):
        off = pl.multiple_of(base + q * QC, 8)
        pltpu.sync_copy(srcs.at[pl.ds(off, QC), :], srcv)
        pltpu.sync_copy(dsts.at[pl.ds(off, QC), :], dstv)
        pltpu.sync_copy(coefs.at[pl.ds(off, QC), :], coefv)
        pltpu.async_copy(h.at[srcv.at[0]], rows0, sem0)

        def _pair(t, c2):
            j0 = 2 * t
            j1 = j0 + 1
            pltpu.async_copy(h.at[srcv.at[j1]], rows1, sem1)
            pltpu.make_async_copy(h.at[srcv.at[j0]], rows0, sem0).wait()
            _scale(rows0, j0)
            pltpu.sync_copy(rows0, acc_sh.at[dstv.at[j0]], add=True)

            j2 = jnp.minimum(j0 + 2, QC - 2)
            pltpu.async_copy(h.at[srcv.at[j2]], rows0, sem0)
            pltpu.make_async_copy(h.at[srcv.at[j1]], rows1, sem1).wait()
            _scale(rows1, j1)
            pltpu.sync_copy(rows1, acc_sh.at[dstv.at[j1]], add=True)
            return c2
        lax.fori_loop(0, QC // 2, _pair, None)
        pltpu.make_async_copy(h.at[srcv.at[0]], rows0, sem0).wait()
        return carry
    lax.fori_loop(0, ng, _quarter, None)
    plsc.subcore_barrier()

    pltpu.sync_copy(acc_sh.at[pl.ds(s * RPT, RPT), :],
                    acc_out.at[c, pl.ds(s * RPT, RPT), :])


def _prop(h, srcs, dsts, coefs):
    f = pl.kernel(
        _prop_body,
        out_type=jax.ShapeDtypeStruct((NC, NP, D), _f32),
        mesh=_mesh(),
        compiler_params=pltpu.CompilerParams(needs_layout_passes=False),
        scratch_types=[
            pltpu.VMEM((QC, C), _i32),
            pltpu.VMEM((QC, C), _i32),
            pltpu.VMEM((QC, C), _f32),
            pltpu.VMEM((C, D), _f32),
            pltpu.VMEM((C, D), _f32),
            pltpu.VMEM_SHARED((NP, D), _f32),
            pltpu.SemaphoreType.DMA,
            pltpu.SemaphoreType.DMA,
        ],
    )
    return f(h, srcs, dsts, coefs)


_BM = 512


def _mm0_body(x_ref, w_ref, o_ref):
    o_ref[...] = jnp.dot(x_ref[...], w_ref[...], preferred_element_type=_f32)


def _mm0(xp, W):
    return pl.pallas_call(
        _mm0_body,
        grid=(NP // _BM,),
        in_specs=[pl.BlockSpec((_BM, D), lambda i: (i, 0)),
                  pl.BlockSpec((D, D), lambda i: (0, 0))],
        out_specs=pl.BlockSpec((_BM, D), lambda i: (i, 0)),
        out_shape=jax.ShapeDtypeStruct((NP, D), _f32),
    )(xp, W)


def _mid_body(a0_ref, a1_ref, h_ref, di_ref, b_ref, w_ref, o_ref):
    d = di_ref[...]
    xb = a0_ref[...] + a1_ref[...] + d * d * h_ref[...] + b_ref[...]
    xb = jnp.maximum(xb, 0.0)
    o_ref[...] = jnp.dot(xb, w_ref[...], preferred_element_type=_f32)


def _mid(a0, a1, h, dinv2d, br, W):
    return pl.pallas_call(
        _mid_body,
        grid=(NP // _BM,),
        in_specs=[pl.BlockSpec((_BM, D), lambda i: (i, 0)),
                  pl.BlockSpec((_BM, D), lambda i: (i, 0)),
                  pl.BlockSpec((_BM, D), lambda i: (i, 0)),
                  pl.BlockSpec((_BM, 1), lambda i: (i, 0)),
                  pl.BlockSpec((1, D), lambda i: (0, 0)),
                  pl.BlockSpec((D, D), lambda i: (0, 0))],
        out_specs=pl.BlockSpec((_BM, D), lambda i: (i, 0)),
        out_shape=jax.ShapeDtypeStruct((NP, D), _f32),
    )(a0, a1, h, dinv2d, br, W)


_BM7 = 256


def _read_body(a0_ref, a1_ref, h_ref, di_ref, b_ref, r0_ref, rb0_ref, r1_ref,
               rb1_ref, bt_ref, o_ref, acc_s, acc_c):
    i = pl.program_id(0)

    @pl.when(i == 0)
    def _():
        acc_s[...] = jnp.zeros_like(acc_s)
        acc_c[...] = jnp.zeros_like(acc_c)

    d = di_ref[...]
    x2 = a0_ref[...] + a1_ref[...] + d * d * h_ref[...] + b_ref[...]
    t = jnp.maximum(
        jnp.dot(x2, r0_ref[...], preferred_element_type=_f32) + rb0_ref[...], 0.0)
    r = jnp.dot(t, r1_ref[...], preferred_element_type=_f32) + rb1_ref[0, 0]
    oh = (bt_ref[...] == lax.broadcasted_iota(_i32, (_BM7, 128), 1)).astype(_f32)
    acc_s[...] += lax.dot_general(oh, r, (((0,), (0,)), ((), ())),
                                  preferred_element_type=_f32)
    acc_c[...] += lax.dot_general(oh, jnp.ones((_BM7, 1), _f32),
                                  (((0,), (0,)), ((), ())),
                                  preferred_element_type=_f32)

    @pl.when(i == pl.num_programs(0) - 1)
    def _():
        o_ref[...] = acc_s[...] / jnp.maximum(acc_c[...], 1.0)


def _read(a0, a1, h, dinv2d, br, R0p, rb0p, R1p, rb1p, bt):
    return pl.pallas_call(
        _read_body,
        grid=(NP // _BM7,),
        in_specs=[pl.BlockSpec((_BM7, D), lambda i: (i, 0)),
                  pl.BlockSpec((_BM7, D), lambda i: (i, 0)),
                  pl.BlockSpec((_BM7, D), lambda i: (i, 0)),
                  pl.BlockSpec((_BM7, 1), lambda i: (i, 0)),
                  pl.BlockSpec((1, D), lambda i: (0, 0)),
                  pl.BlockSpec((D, D), lambda i: (0, 0)),
                  pl.BlockSpec((1, D), lambda i: (0, 0)),
                  pl.BlockSpec((D, 1), lambda i: (0, 0)),
                  pl.BlockSpec((1, 1), lambda i: (0, 0)),
                  pl.BlockSpec((_BM7, 1), lambda i: (i, 0))],
        out_specs=pl.BlockSpec((128, 1), lambda i: (0, 0)),
        out_shape=jax.ShapeDtypeStruct((128, 1), _f32),
        scratch_shapes=[pltpu.VMEM((128, 1), _f32),
                        pltpu.VMEM((128, 1), _f32)],
    )(a0, a1, h, dinv2d, br, R0p, rb0p, R1p, rb1p, bt)


def kernel(x, edge_index, edge_attr, batch, W0, b0, W1, b1, W2, b2, R0, rb0, R1, rb1):
    src = edge_index[0]
    dst = edge_index[1]
    src_p = jnp.pad(src, (0, EP - E))
    dst_p = jnp.pad(dst, (0, EP - E))
    srcs = src_p.reshape(NW, CH, C)
    dsts = dst_p.reshape(NW, CH, C)
    ews = jnp.pad(edge_attr, (0, EP - E)).reshape(NW, CH, C)
    srcs_r = src_p.reshape(EB, 128)
    dsts_r = dst_p.reshape(EB, 128)
    xp = jnp.pad(x, ((0, NP - N), (0, 0)))
    bt = jnp.pad(batch, (0, NP - N), constant_values=127).reshape(NP, 1)
    R0p = jnp.pad(R0, ((0, 0), (0, 128 - R0.shape[1])))
    rb0p = jnp.pad(rb0, (0, 128 - rb0.shape[0])).reshape(1, 128)
    R1p = jnp.pad(R1, ((0, 128 - R1.shape[0]), (0, 0)))
    rb1p = rb1.reshape(1, 1)
    b0r = b0.reshape(1, D)
    b1r = b1.reshape(1, D)
    b2r = b2.reshape(1, D)

    dinv, coef = _pre(dsts, ews, srcs)
    dinv2d = dinv.reshape(NP, 1)
    coef_r = coef.reshape(EB, 128)

    h0 = _mm0(xp, W0)
    acc = _prop(h0, srcs_r, dsts_r, coef_r)
    h1 = _mid(acc[0], acc[1], h0, dinv2d, b0r, W1)
    acc = _prop(h1, srcs_r, dsts_r, coef_r)
    h2 = _mid(acc[0], acc[1], h1, dinv2d, b1r, W2)
    acc = _prop(h2, srcs_r, dsts_r, coef_r)
    outp = _read(acc[0], acc[1], h2, dinv2d, b2r, R0p, rb0p, R1p, rb1p, bt)
    return outp[:G]


# speedup vs baseline: 1.2288x; 1.1609x over previous
import jax
import jax.numpy as jnp
from jax import lax
from jax.experimental import pallas as pl
from jax.experimental.pallas import tpu as pltpu
from jax.experimental.pallas import tpu_sc as plsc

N = 10000
E = 320000
D = 128
G = 64

NC, NS = 2, 16
NW = NC * NS
NP = 10240
RPT = NP // NS
CH, C = 80, 128
QC = 8
EW = CH * C
EP = NW * EW
EB = EP // 128
N0 = 112
N1 = 160 - N0
R0B = NS * N0

_f32 = jnp.float32
_i32 = jnp.int32


def _mesh():
    return plsc.VectorSubcoreMesh(core_axis_name="c", subcore_axis_name="s",
                                  num_cores=NC, num_subcores=NS)


def _zero16(ref, ngroups):
    def _z(g, carry):
        ref[g // 8, pl.ds((g % 8) * 16, 16)] = jnp.zeros((16,), _f32)
        return carry
    lax.fori_loop(0, ngroups, _z, None)


def _pre_body(dsts, ews, srcs, dinv_out, coef_out,
              dstv, ewv, srcv, degp, sumb, dinvc, dinv_full,
              coefv, sh_deg, sh_dinv):
    c = lax.axis_index("c")
    s = lax.axis_index("s")

    def _zd(i, carry):
        degp[pl.ds(i * 16, 16)] = jnp.zeros((16,), _f32)
        return carry
    lax.fori_loop(0, NP // 16, _zd, None)

    def _slice(k):
        pltpu.sync_copy(dsts.at[k], dstv)
        pltpu.sync_copy(ews.at[k], ewv)
        def _g(g, carry):
            r, col = g // 8, (g % 8) * 16
            d16 = dstv[r, pl.ds(col, 16)]
            w16 = ewv[r, pl.ds(col, 16)]
            plsc.addupdate_scatter(degp, [d16], w16)
            return carry
        lax.fori_loop(0, CH * 8, _g, None)
    _slice(2 * s)
    _slice(2 * s + 1)

    pltpu.sync_copy(degp, sh_deg.at[s])
    plsc.subcore_barrier()

    pltpu.sync_copy(sh_deg.at[:, pl.ds(s * RPT, RPT)], sumb)
    def _rs(v, carry):
        dsum = jnp.zeros((16,), _f32)
        for r in range(NS):
            dsum = dsum + sumb[r, pl.ds(v * 16, 16)]
        dsum = dsum + 1.0
        i = plsc.bitcast(dsum, _i32)
        i = 0x5F3759DF - lax.shift_right_logical(i, 1)
        y = plsc.bitcast(i, _f32)
        for _ in range(3):
            y = y * (1.5 - 0.5 * dsum * y * y)
        dinvc[pl.ds(v * 16, 16)] = y
        return carry
    lax.fori_loop(0, 40, _rs, None)
    pltpu.sync_copy(dinvc, sh_dinv.at[pl.ds(s * RPT, RPT)])

    @pl.when(c == 0)
    def _():
        pltpu.sync_copy(dinvc, dinv_out.at[pl.ds(s * RPT, RPT)])
    plsc.subcore_barrier()

    pltpu.sync_copy(sh_dinv, dinv_full)
    wid = c * NS + s
    pltpu.sync_copy(srcs.at[wid], srcv)
    pltpu.sync_copy(dsts.at[wid], dstv)
    pltpu.sync_copy(ews.at[wid], ewv)
    def _ce(g, carry):
        r, col = g // 8, (g % 8) * 16
        s16 = srcv[r, pl.ds(col, 16)]
        d16 = dstv[r, pl.ds(col, 16)]
        w16 = ewv[r, pl.ds(col, 16)]
        cc = w16 * plsc.load_gather(dinv_full, [s16]) * plsc.load_gather(dinv_full, [d16])
        coefv[r, pl.ds(col, 16)] = cc
        return carry
    lax.fori_loop(0, CH * 8, _ce, None)
    pltpu.sync_copy(coefv, coef_out.at[wid])


def _pre(dsts, ews, srcs):
    f = pl.kernel(
        _pre_body,
        out_type=(jax.ShapeDtypeStruct((NP,), _f32),
                  jax.ShapeDtypeStruct((NW, CH, C), _f32)),
        mesh=_mesh(),
        compiler_params=pltpu.CompilerParams(needs_layout_passes=False),
        scratch_types=[
            pltpu.VMEM((CH, C), _i32),
            pltpu.VMEM((CH, C), _f32),
            pltpu.VMEM((CH, C), _i32),
            pltpu.VMEM((NP,), _f32),
            pltpu.VMEM((NS, RPT), _f32),
            pltpu.VMEM((RPT,), _f32),
            pltpu.VMEM((NP,), _f32),
            pltpu.VMEM((CH, C), _f32),
            pltpu.VMEM_SHARED((NS, NP), _f32),
            pltpu.VMEM_SHARED((NP,), _f32),
        ],
    )
    return f(dsts, ews, srcs)


def _prop_body(h, srcs, dsts, coefs, acc_out,
               srcv, dstv, coefv, rows0, rows1, acc_sh, sem0, sem1):
    c = lax.axis_index("c")
    s = lax.axis_index("s")

    _zero16(rows0, C * 8)
    for k in range(RPT // C):
        pltpu.sync_copy(rows0, acc_sh.at[pl.ds(s * RPT + k * C, C), :])
    plsc.subcore_barrier()

    def _scale(rows, j):
        def _grp(g8, c2):
            c16 = coefv[j, pl.ds(g8 * 16, 16)]
            for r16 in range(16):
                e = g8 * 16 + r16
                bc = c16.at[jnp.full((16,), r16, _i32)].get(
                    mode="promise_in_bounds")
                for k in range(8):
                    sl = pl.ds(k * 16, 16)
                    rows[e, sl] = rows[e, sl] * bc
            return c2
        lax.fori_loop(0, 8, _grp, None)

    ng = jnp.where(c == 0, N0 // QC, N1 // QC)
    base = jnp.where(c == 0, s * N0, R0B + s * N1)

    def _quarter(q, carry):
        off = pl.multiple_of(base + q * QC, 8)
        pltpu.sync_copy(srcs.at[pl.ds(off, QC), :], srcv)
        pltpu.sync_copy(dsts.at[pl.ds(off, QC), :], dstv)
        pltpu.sync_copy(coefs.at[pl.ds(off, QC), :], coefv)
        pltpu.async_copy(h.at[srcv.at[0]], rows0, sem0)

        def _pair(t, c2):
            j0 = 2 * t
            j1 = j0 + 1
            pltpu.async_copy(h.at[srcv.at[j1]], rows1, sem1)
            pltpu.make_async_copy(h.at[srcv.at[j0]], rows0, sem0).wait()
            _scale(rows0, j0)
            pltpu.sync_copy(rows0, acc_sh.at[dstv.at[j0]], add=True)

            j2 = jnp.minimum(j0 + 2, QC - 2)
            pltpu.async_copy(h.at[srcv.at[j2]], rows0, sem0)
            pltpu.make_async_copy(h.at[srcv.at[j1]], rows1, sem1).wait()
            _scale(rows1, j1)
            pltpu.sync_copy(rows1, acc_sh.at[dstv.at[j1]], add=True)
            return c2
        lax.fori_loop(0, QC // 2, _pair, None)
        pltpu.make_async_copy(h.at[srcv.at[0]], rows0, sem0).wait()
        return carry
    lax.fori_loop(0, ng, _quarter, None)
    plsc.subcore_barrier()

    pltpu.sync_copy(acc_sh.at[pl.ds(s * RPT, RPT), :],
                    acc_out.at[c, pl.ds(s * RPT, RPT), :])


def _prop(h, srcs, dsts, coefs):
    f = pl.kernel(
        _prop_body,
        out_type=jax.ShapeDtypeStruct((NC, NP, D), _f32),
        mesh=_mesh(),
        compiler_params=pltpu.CompilerParams(needs_layout_passes=False),
        scratch_types=[
            pltpu.VMEM((QC, C), _i32),
            pltpu.VMEM((QC, C), _i32),
            pltpu.VMEM((QC, C), _f32),
            pltpu.VMEM((C, D), _f32),
            pltpu.VMEM((C, D), _f32),
            pltpu.VMEM_SHARED((NP, D), _f32),
            pltpu.SemaphoreType.DMA,
            pltpu.SemaphoreType.DMA,
        ],
    )
    return f(h, srcs, dsts, coefs)


_BM = 512


def _mm0_body(x_ref, w_ref, o_ref):
    o_ref[...] = jnp.dot(x_ref[...], w_ref[...], preferred_element_type=_f32)


def _mm0(xp, W):
    return pl.pallas_call(
        _mm0_body,
        grid=(NP // _BM,),
        in_specs=[pl.BlockSpec((_BM, D), lambda i: (i, 0)),
                  pl.BlockSpec((D, D), lambda i: (0, 0))],
        out_specs=pl.BlockSpec((_BM, D), lambda i: (i, 0)),
        out_shape=jax.ShapeDtypeStruct((NP, D), _f32),
    )(xp, W)


def _mid_body(a0_ref, a1_ref, h_ref, di_ref, b_ref, w_ref, o_ref):
    d = di_ref[...]
    xb = a0_ref[...] + a1_ref[...] + d * d * h_ref[...] + b_ref[...]
    xb = jnp.maximum(xb, 0.0)
    o_ref[...] = jnp.dot(xb, w_ref[...], preferred_element_type=_f32)


def _mid(a0, a1, h, dinv2d, br, W):
    return pl.pallas_call(
        _mid_body,
        grid=(NP // _BM,),
        in_specs=[pl.BlockSpec((_BM, D), lambda i: (i, 0)),
                  pl.BlockSpec((_BM, D), lambda i: (i, 0)),
                  pl.BlockSpec((_BM, D), lambda i: (i, 0)),
                  pl.BlockSpec((_BM, 1), lambda i: (i, 0)),
                  pl.BlockSpec((1, D), lambda i: (0, 0)),
                  pl.BlockSpec((D, D), lambda i: (0, 0))],
        out_specs=pl.BlockSpec((_BM, D), lambda i: (i, 0)),
        out_shape=jax.ShapeDtypeStruct((NP, D), _f32),
    )(a0, a1, h, dinv2d, br, W)


_BM7 = 256


def _read_body(a0_ref, a1_ref, h_ref, di_ref, b_ref, r0_ref, rb0_ref, r1_ref,
               rb1_ref, bt_ref, o_ref, acc_s, acc_c):
    i = pl.program_id(0)

    @pl.when(i == 0)
    def _():
        acc_s[...] = jnp.zeros_like(acc_s)
        acc_c[...] = jnp.zeros_like(acc_c)

    d = di_ref[...]
    x2 = a0_ref[...] + a1_ref[...] + d * d * h_ref[...] + b_ref[...]
    t = jnp.maximum(
        jnp.dot(x2, r0_ref[...], preferred_element_type=_f32) + rb0_ref[...], 0.0)
    r = jnp.dot(t, r1_ref[...], preferred_element_type=_f32) + rb1_ref[0, 0]
    oh = (bt_ref[...] == lax.broadcasted_iota(_i32, (_BM7, 128), 1)).astype(_f32)
    acc_s[...] += lax.dot_general(oh, r, (((0,), (0,)), ((), ())),
                                  preferred_element_type=_f32)
    acc_c[...] += lax.dot_general(oh, jnp.ones((_BM7, 1), _f32),
                                  (((0,), (0,)), ((), ())),
                                  preferred_element_type=_f32)

    @pl.when(i == pl.num_programs(0) - 1)
    def _():
        o_ref[...] = acc_s[...] / jnp.maximum(acc_c[...], 1.0)


def _read(a0, a1, h, dinv2d, br, R0p, rb0p, R1p, rb1p, bt):
    return pl.pallas_call(
        _read_body,
        grid=(NP // _BM7,),
        in_specs=[pl.BlockSpec((_BM7, D), lambda i: (i, 0)),
                  pl.BlockSpec((_BM7, D), lambda i: (i, 0)),
                  pl.BlockSpec((_BM7, D), lambda i: (i, 0)),
                  pl.BlockSpec((_BM7, 1), lambda i: (i, 0)),
                  pl.BlockSpec((1, D), lambda i: (0, 0)),
                  pl.BlockSpec((D, D), lambda i: (0, 0)),
                  pl.BlockSpec((1, D), lambda i: (0, 0)),
                  pl.BlockSpec((D, 1), lambda i: (0, 0)),
                  pl.BlockSpec((1, 1), lambda i: (0, 0)),
                  pl.BlockSpec((_BM7, 1), lambda i: (i, 0))],
        out_specs=pl.BlockSpec((128, 1), lambda i: (0, 0)),
        out_shape=jax.ShapeDtypeStruct((128, 1), _f32),
        scratch_shapes=[pltpu.VMEM((128, 1), _f32),
                        pltpu.VMEM((128, 1), _f32)],
    )(a0, a1, h, dinv2d, br, R0p, rb0p, R1p, rb1p, bt)


def kernel(x, edge_index, edge_attr, batch, W0, b0, W1, b1, W2, b2, R0, rb0, R1, rb1):
    src = edge_index[0]
    dst = edge_index[1]
    src_p = jnp.pad(src, (0, EP - E))
    dst_p = jnp.pad(dst, (0, EP - E))
    srcs = src_p.reshape(NW, CH, C)
    dsts = dst_p.reshape(NW, CH, C)
    ews = jnp.pad(edge_attr, (0, EP - E)).reshape(NW, CH, C)
    srcs_r = src_p.reshape(EB, 128)
    dsts_r = dst_p.reshape(EB, 128)
    xp = jnp.pad(x, ((0, NP - N), (0, 0)))
    bt = jnp.pad(batch, (0, NP - N), constant_values=127).reshape(NP, 1)
    R0p = jnp.pad(R0, ((0, 0), (0, 128 - R0.shape[1])))
    rb0p = jnp.pad(rb0, (0, 128 - rb0.shape[0])).reshape(1, 128)
    R1p = jnp.pad(R1, ((0, 128 - R1.shape[0]), (0, 0)))
    rb1p = rb1.reshape(1, 1)
    b0r = b0.reshape(1, D)
    b1r = b1.reshape(1, D)
    b2r = b2.reshape(1, D)

    dinv, coef = _pre(dsts, ews, srcs)
    dinv2d = dinv.reshape(NP, 1)
    coef_r = coef.reshape(EB, 128)

    h0 = _mm0(xp, W0)
    acc = _prop(h0, srcs_r, dsts_r, coef_r)
    h1 = _mid(acc[0], acc[1], h0, dinv2d, b0r, W1)
    acc = _prop(h1, srcs_r, dsts_r, coef_r)
    h2 = _mid(acc[0], acc[1], h1, dinv2d, b1r, W2)
    acc = _prop(h2, srcs_r, dsts_r, coef_r)
    outp = _read(acc[0], acc[1], h2, dinv2d, b2r, R0p, rb0p, R1p, rb1p, bt)
    return outp[:G]
